# baseline (speedup 1.0000x reference)
import functools

import jax
import jax.numpy as jnp
from jax import lax
from jax.experimental import pallas as pl
from jax.experimental.pallas import tpu as pltpu
from jax.experimental.pallas import tpu_sc as plsc

_B = 16384
_V = 1000000
_D = 64
_H1 = 128
_H2 = 64
_NW = 32
_BPW = _B // _NW
_G16 = _BPW // 16

_BLKT = 16384
_BLK = 1024


def _convert_body(ut_ref, it_ref, eyea_ref, eyeb_ref, o_ref):
    y = lax.dot_general(ut_ref[...], eyea_ref[...], (((0,), (0,)), ((), ())),
                        preferred_element_type=jnp.float32)
    y = y + lax.dot_general(it_ref[...], eyeb_ref[...],
                            (((0,), (0,)), ((), ())),
                            preferred_element_type=jnp.float32)
    o_ref[...] = y


def _convert_tables(uT, iT):
    eyea = jnp.concatenate(
        [jnp.eye(_D, dtype=jnp.float32),
         jnp.zeros((_D, _D), jnp.float32)], axis=1)
    eyeb = jnp.concatenate(
        [jnp.zeros((_D, _D), jnp.float32),
         jnp.eye(_D, dtype=jnp.float32)], axis=1)
    nblk = pl.cdiv(_V, _BLKT)
    return pl.pallas_call(
        _convert_body,
        grid=(nblk,),
        in_specs=[
            pl.BlockSpec((_D, _BLKT), lambda g: (0, g)),
            pl.BlockSpec((_D, _BLKT), lambda g: (0, g)),
            pl.BlockSpec((_D, 128), lambda g: (0, 0)),
            pl.BlockSpec((_D, 128), lambda g: (0, 0)),
        ],
        out_specs=pl.BlockSpec((_BLKT, 128), lambda g: (g, 0)),
        out_shape=jax.ShapeDtypeStruct((_V, 128), jnp.float32),
        compiler_params=pltpu.CompilerParams(
            dimension_semantics=("arbitrary",)),
    )(uT, iT, eyea, eyeb)


def _sc_gather_rows(uidx3, iidx3, tab):
    mesh = plsc.VectorSubcoreMesh(core_axis_name="c", subcore_axis_name="s")

    @functools.partial(
        pl.kernel,
        mesh=mesh,
        out_type=(
            jax.ShapeDtypeStruct((_B, 128), jnp.float32),
            jax.ShapeDtypeStruct((_B, 128), jnp.float32),
        ),
        scratch_types=[
            pltpu.VMEM((1, _BPW), jnp.int32),
            pltpu.VMEM((1, _BPW), jnp.int32),
            pltpu.VMEM((_BPW, 128), jnp.float32),
            pltpu.SemaphoreType.DMA,
        ],
        compiler_params=pltpu.CompilerParams(use_tc_tiling_on_sc=True),
    )
    def k(uidx_hbm, iidx_hbm, tab_hbm,
          urows_out, vrows_out, uidx_v, iidx_v, rows_v, sem):
        wid = lax.axis_index("s") * 2 + lax.axis_index("c")
        base = wid * _BPW
        pltpu.sync_copy(uidx_hbm.at[wid], uidx_v)
        pltpu.sync_copy(iidx_hbm.at[wid], iidx_v)

        for idx_v, out in ((uidx_v, urows_out), (iidx_v, vrows_out)):
            copies = []
            for g in range(_G16):
                vec = idx_v.at[0][pl.ds(g * 16, 16)]
                copies.append(pltpu.async_copy(
                    tab_hbm.at[vec], rows_v.at[pl.ds(g * 16, 16), :], sem))
            for cp in copies:
                cp.wait()
            pltpu.sync_copy(rows_v, out.at[pl.ds(base, _BPW)])

    return k(uidx3, iidx3, tab)


def _sc_gather_bias(uidx2, iidx2, ubias1, ibias1):
    mesh = plsc.VectorSubcoreMesh(core_axis_name="c", subcore_axis_name="s")
    _CPW = 4

    @functools.partial(
        pl.kernel,
        mesh=mesh,
        out_type=(
            jax.ShapeDtypeStruct((_B // 128, 128), jnp.float32),
            jax.ShapeDtypeStruct((_B // 128, 128), jnp.float32),
        ),
        scratch_types=[
            pltpu.VMEM((_CPW, 128), jnp.int32),
            pltpu.VMEM((_CPW, 128), jnp.int32),
            pltpu.VMEM((_CPW, 128), jnp.float32),
            pltpu.VMEM((_CPW, 128), jnp.float32),
            pltpu.SemaphoreType.DMA,
        ],
        compiler_params=pltpu.CompilerParams(use_tc_tiling_on_sc=False),
    )
    def k(uidx_hbm, iidx_hbm, ub_hbm, ib_hbm,
          ub_out, ib_out, uidx_v, iidx_v, ub_v, ib_v, sem):
        wid = lax.axis_index("s") * 2 + lax.axis_index("c")
        cbase = wid * _CPW
        pltpu.sync_copy(uidx_hbm.at[pl.ds(cbase, _CPW)], uidx_v)
        pltpu.sync_copy(iidx_hbm.at[pl.ds(cbase, _CPW)], iidx_v)
        copies = []
        for j in range(_CPW):
            copies.append(pltpu.async_copy(ub_hbm.at[uidx_v.at[j]], ub_v.at[j], sem))
            copies.append(pltpu.async_copy(ib_hbm.at[iidx_v.at[j]], ib_v.at[j], sem))
        for c in copies:
            c.wait()
        pltpu.sync_copy(ub_v, ub_out.at[pl.ds(cbase, _CPW)])
        pltpu.sync_copy(ib_v, ib_out.at[pl.ds(cbase, _CPW)])

    return k(uidx2, iidx2, ubias1, ibias1)


def _mlp_body(u_ref, v_ref, ub_ref, ib_ref, w1a_ref, w1b_ref, b1_ref,
              w2_ref, b2_ref, w3_ref, b3_ref, o_ref):
    u = u_ref[:, :_D]
    v = v_ref[:, _D:]
    h1 = jnp.dot(u, w1a_ref[...], preferred_element_type=jnp.float32)
    h1 = h1 + jnp.dot(v, w1b_ref[...], preferred_element_type=jnp.float32)
    h1 = jnp.maximum(h1 + b1_ref[...], 0.0)
    h2 = jnp.dot(h1, w2_ref[...], preferred_element_type=jnp.float32)
    h2 = jnp.maximum(h2 + b2_ref[...], 0.0)
    pred = jnp.sum(h2 * w3_ref[...], axis=1)
    o_ref[...] = pred + b3_ref[0] + ub_ref[...] + ib_ref[...]


def _tc_mlp(u, v, ub, ib, w1aT, w1bT, b1, w2T, b2, w3, b3):
    grid = (_B // _BLK,)
    return pl.pallas_call(
        _mlp_body,
        grid=grid,
        in_specs=[
            pl.BlockSpec((_BLK, 128), lambda i: (i, 0)),
            pl.BlockSpec((_BLK, 128), lambda i: (i, 0)),
            pl.BlockSpec((_BLK,), lambda i: (i,)),
            pl.BlockSpec((_BLK,), lambda i: (i,)),
            pl.BlockSpec((_D, _H1), lambda i: (0, 0)),
            pl.BlockSpec((_D, _H1), lambda i: (0, 0)),
            pl.BlockSpec((_H1,), lambda i: (0,)),
            pl.BlockSpec((_H1, _H2), lambda i: (0, 0)),
            pl.BlockSpec((_H2,), lambda i: (0,)),
            pl.BlockSpec((1, _H2), lambda i: (0, 0)),
            pl.BlockSpec(memory_space=pltpu.SMEM),
        ],
        out_specs=pl.BlockSpec((_BLK,), lambda i: (i,)),
        out_shape=jax.ShapeDtypeStruct((_B,), jnp.float32),
        compiler_params=pltpu.CompilerParams(
            dimension_semantics=("parallel",)),
    )(u, v, ub, ib, w1aT, w1bT, b1, w2T, b2, w3, b3)


def kernel(user_idx, item_idx, user_emb, item_emb, user_bias, item_bias,
           W1, b1, W2, b2, W3, b3):
    uidx = user_idx.astype(jnp.int32)
    iidx = item_idx.astype(jnp.int32)
    tab = _convert_tables(user_emb.T, item_emb.T)
    urows, vrows = _sc_gather_rows(
        uidx.reshape(_NW, 1, _BPW), iidx.reshape(_NW, 1, _BPW), tab)
    ubg, ibg = _sc_gather_bias(
        uidx.reshape(_B // 128, 128), iidx.reshape(_B // 128, 128),
        user_bias.reshape(-1), item_bias.reshape(-1))
    w1aT = W1[:, :_D].T
    w1bT = W1[:, _D:].T
    return _tc_mlp(urows, vrows, ubg.reshape(_B), ibg.reshape(_B),
                   w1aT, w1bT, b1, W2.T, b2, W3, b3)


# speedup vs baseline: 1.0204x; 1.0204x over previous
import functools

import jax
import jax.numpy as jnp
from jax import lax
from jax.experimental import pallas as pl
from jax.experimental.pallas import tpu as pltpu
from jax.experimental.pallas import tpu_sc as plsc

_B = 16384
_V = 1000000
_D = 64
_H1 = 128
_H2 = 64
_NW = 32
_BPW = _B // _NW
_G16 = _BPW // 16

_BLKT = 20480
_BLK = 1024


def _convert_body(ut_ref, it_ref, eyea_ref, eyeb_ref, o_ref):
    y = lax.dot_general(ut_ref[...], eyea_ref[...], (((0,), (0,)), ((), ())),
                        preferred_element_type=jnp.float32)
    y = y + lax.dot_general(it_ref[...], eyeb_ref[...],
                            (((0,), (0,)), ((), ())),
                            preferred_element_type=jnp.float32)
    o_ref[...] = y


def _convert_tables(uT, iT):
    eyea = jnp.concatenate(
        [jnp.eye(_D, dtype=jnp.float32),
         jnp.zeros((_D, _D), jnp.float32)], axis=1)
    eyeb = jnp.concatenate(
        [jnp.zeros((_D, _D), jnp.float32),
         jnp.eye(_D, dtype=jnp.float32)], axis=1)
    nblk = pl.cdiv(_V, _BLKT)
    return pl.pallas_call(
        _convert_body,
        grid=(nblk,),
        in_specs=[
            pl.BlockSpec((_D, _BLKT), lambda g: (0, g)),
            pl.BlockSpec((_D, _BLKT), lambda g: (0, g)),
            pl.BlockSpec((_D, 128), lambda g: (0, 0)),
            pl.BlockSpec((_D, 128), lambda g: (0, 0)),
        ],
        out_specs=pl.BlockSpec((_BLKT, 128), lambda g: (g, 0)),
        out_shape=jax.ShapeDtypeStruct((_V, 128), jnp.float32),
        compiler_params=pltpu.CompilerParams(
            dimension_semantics=("arbitrary",)),
    )(uT, iT, eyea, eyeb)


def _sc_gather_rows(uidx3, iidx3, tab):
    mesh = plsc.VectorSubcoreMesh(core_axis_name="c", subcore_axis_name="s")

    @functools.partial(
        pl.kernel,
        mesh=mesh,
        out_type=(
            jax.ShapeDtypeStruct((_B, 128), jnp.float32),
            jax.ShapeDtypeStruct((_B, 128), jnp.float32),
        ),
        scratch_types=[
            pltpu.VMEM((1, _BPW), jnp.int32),
            pltpu.VMEM((1, _BPW), jnp.int32),
            pltpu.VMEM((_BPW, 128), jnp.float32),
            pltpu.SemaphoreType.DMA,
        ],
        compiler_params=pltpu.CompilerParams(use_tc_tiling_on_sc=True),
    )
    def k(uidx_hbm, iidx_hbm, tab_hbm,
          urows_out, vrows_out, uidx_v, iidx_v, rows_v, sem):
        wid = lax.axis_index("s") * 2 + lax.axis_index("c")
        base = wid * _BPW
        pltpu.sync_copy(uidx_hbm.at[wid], uidx_v)
        pltpu.sync_copy(iidx_hbm.at[wid], iidx_v)

        for idx_v, out in ((uidx_v, urows_out), (iidx_v, vrows_out)):
            copies = []
            for g in range(_G16):
                vec = idx_v.at[0][pl.ds(g * 16, 16)]
                copies.append(pltpu.async_copy(
                    tab_hbm.at[vec], rows_v.at[pl.ds(g * 16, 16), :], sem))
            for cp in copies:
                cp.wait()
            pltpu.sync_copy(rows_v, out.at[pl.ds(base, _BPW)])

    return k(uidx3, iidx3, tab)


def _sc_gather_bias(uidx2, iidx2, ubias1, ibias1):
    mesh = plsc.VectorSubcoreMesh(core_axis_name="c", subcore_axis_name="s")
    _CPW = 4

    @functools.partial(
        pl.kernel,
        mesh=mesh,
        out_type=(
            jax.ShapeDtypeStruct((_B // 128, 128), jnp.float32),
            jax.ShapeDtypeStruct((_B // 128, 128), jnp.float32),
        ),
        scratch_types=[
            pltpu.VMEM((_CPW, 128), jnp.int32),
            pltpu.VMEM((_CPW, 128), jnp.int32),
            pltpu.VMEM((_CPW, 128), jnp.float32),
            pltpu.VMEM((_CPW, 128), jnp.float32),
            pltpu.SemaphoreType.DMA,
        ],
        compiler_params=pltpu.CompilerParams(use_tc_tiling_on_sc=False),
    )
    def k(uidx_hbm, iidx_hbm, ub_hbm, ib_hbm,
          ub_out, ib_out, uidx_v, iidx_v, ub_v, ib_v, sem):
        wid = lax.axis_index("s") * 2 + lax.axis_index("c")
        cbase = wid * _CPW
        pltpu.sync_copy(uidx_hbm.at[pl.ds(cbase, _CPW)], uidx_v)
        pltpu.sync_copy(iidx_hbm.at[pl.ds(cbase, _CPW)], iidx_v)
        copies = []
        for j in range(_CPW):
            copies.append(pltpu.async_copy(ub_hbm.at[uidx_v.at[j]], ub_v.at[j], sem))
            copies.append(pltpu.async_copy(ib_hbm.at[iidx_v.at[j]], ib_v.at[j], sem))
        for c in copies:
            c.wait()
        pltpu.sync_copy(ub_v, ub_out.at[pl.ds(cbase, _CPW)])
        pltpu.sync_copy(ib_v, ib_out.at[pl.ds(cbase, _CPW)])

    return k(uidx2, iidx2, ubias1, ibias1)


def _mlp_body(u_ref, v_ref, ub_ref, ib_ref, w1a_ref, w1b_ref, b1_ref,
              w2_ref, b2_ref, w3_ref, b3_ref, o_ref):
    u = u_ref[:, :_D]
    v = v_ref[:, _D:]
    h1 = jnp.dot(u, w1a_ref[...], preferred_element_type=jnp.float32)
    h1 = h1 + jnp.dot(v, w1b_ref[...], preferred_element_type=jnp.float32)
    h1 = jnp.maximum(h1 + b1_ref[...], 0.0)
    h2 = jnp.dot(h1, w2_ref[...], preferred_element_type=jnp.float32)
    h2 = jnp.maximum(h2 + b2_ref[...], 0.0)
    pred = jnp.sum(h2 * w3_ref[...], axis=1)
    o_ref[...] = pred + b3_ref[0] + ub_ref[...] + ib_ref[...]


def _tc_mlp(u, v, ub, ib, w1aT, w1bT, b1, w2T, b2, w3, b3):
    grid = (_B // _BLK,)
    return pl.pallas_call(
        _mlp_body,
        grid=grid,
        in_specs=[
            pl.BlockSpec((_BLK, 128), lambda i: (i, 0)),
            pl.BlockSpec((_BLK, 128), lambda i: (i, 0)),
            pl.BlockSpec((_BLK,), lambda i: (i,)),
            pl.BlockSpec((_BLK,), lambda i: (i,)),
            pl.BlockSpec((_D, _H1), lambda i: (0, 0)),
            pl.BlockSpec((_D, _H1), lambda i: (0, 0)),
            pl.BlockSpec((_H1,), lambda i: (0,)),
            pl.BlockSpec((_H1, _H2), lambda i: (0, 0)),
            pl.BlockSpec((_H2,), lambda i: (0,)),
            pl.BlockSpec((1, _H2), lambda i: (0, 0)),
            pl.BlockSpec(memory_space=pltpu.SMEM),
        ],
        out_specs=pl.BlockSpec((_BLK,), lambda i: (i,)),
        out_shape=jax.ShapeDtypeStruct((_B,), jnp.float32),
        compiler_params=pltpu.CompilerParams(
            dimension_semantics=("parallel",)),
    )(u, v, ub, ib, w1aT, w1bT, b1, w2T, b2, w3, b3)


def kernel(user_idx, item_idx, user_emb, item_emb, user_bias, item_bias,
           W1, b1, W2, b2, W3, b3):
    uidx = user_idx.astype(jnp.int32)
    iidx = item_idx.astype(jnp.int32)
    tab = _convert_tables(user_emb.T, item_emb.T)
    urows, vrows = _sc_gather_rows(
        uidx.reshape(_NW, 1, _BPW), iidx.reshape(_NW, 1, _BPW), tab)
    ubg, ibg = _sc_gather_bias(
        uidx.reshape(_B // 128, 128), iidx.reshape(_B // 128, 128),
        user_bias.reshape(-1), item_bias.reshape(-1))
    w1aT = W1[:, :_D].T
    w1bT = W1[:, _D:].T
    return _tc_mlp(urows, vrows, ubg.reshape(_B), ibg.reshape(_B),
                   w1aT, w1bT, b1, W2.T, b2, W3, b3)
